# Initial kernel scaffold; baseline (speedup 1.0000x reference)
import jax
import jax.numpy as jnp
from jax.experimental import pallas as pl


def kernel(indices, table):
    raise NotImplementedError("write your pallas kernel here")


# speedup vs baseline: 1.8778x; 1.8778x over previous
import functools

import jax
import jax.numpy as jnp
from jax import lax
from jax.experimental import pallas as pl
from jax.experimental.pallas import tpu as pltpu
from jax.experimental.pallas import tpu_sc as plsc

_BATCH = 16384
_HIST = 50
_EMBED = 64
_B = _BATCH * _HIST

_info = plsc.get_sparse_core_info()
_NC, _NS = _info.num_cores, _info.num_subcores
_NW = _NC * _NS
_BPW = _B // _NW
_CH = 128
_NCHUNK = _BPW // _CH
_NB = 4

_mesh = plsc.VectorSubcoreMesh(core_axis_name="c", subcore_axis_name="s")


@functools.partial(
    pl.kernel,
    mesh=_mesh,
    out_type=jax.ShapeDtypeStruct((_NW, _NCHUNK, _CH, _EMBED), jnp.float32),
    scratch_types=[
        pltpu.VMEM((_NCHUNK, _CH), jnp.int32),
    ]
    + [pltpu.VMEM((_CH, _EMBED), jnp.float32) for _ in range(_NB)]
    + [pltpu.SemaphoreType.DMA for _ in range(_NB)],
    compiler_params=pltpu.CompilerParams(use_tc_tiling_on_sc=False),
)
def _gather_sc(idx_hbm, table_hbm, out_hbm, idx_v, *bufs_and_sems):
    rows = bufs_and_sems[:_NB]
    sems = bufs_and_sems[_NB:]
    wid = lax.axis_index("s") * _NC + lax.axis_index("c")
    pltpu.sync_copy(idx_hbm.at[wid], idx_v)

    for b in range(_NB):
        pltpu.async_copy(table_hbm.at[idx_v.at[b]], rows[b], sems[b])

    def body(c0):
        for b in range(_NB):
            c = c0 + b
            pltpu.make_async_copy(
                table_hbm.at[idx_v.at[c]], rows[b], sems[b]
            ).wait()
            pltpu.sync_copy(rows[b], out_hbm.at[wid, c])

            @pl.when(c + _NB < _NCHUNK)
            def _():
                pltpu.async_copy(
                    table_hbm.at[idx_v.at[c + _NB]], rows[b], sems[b]
                )

    pl.loop(0, _NCHUNK, step=_NB)(body)


def kernel(indices, table):
    idx = indices.astype(jnp.int32).reshape(_NW, _NCHUNK, _CH)
    out = _gather_sc(idx, table)
    return out.reshape(_BATCH, _HIST, _EMBED)
